# Initial kernel scaffold; baseline (speedup 1.0000x reference)
import jax
import jax.numpy as jnp
from jax.experimental import pallas as pl


def kernel(input, rim_hidden, Wq, Wk, Wv):
    raise NotImplementedError("write your pallas kernel here")


# speedup vs baseline: 2.4197x; 2.4197x over previous
import jax
import jax.numpy as jnp
from jax.experimental import pallas as pl

_ACT = 2


def _prep_kernel(h_ref, wq_ref, wk_ref, w_ref):
    h = h_ref[0]
    q = jnp.dot(h, wq_ref[0], preferred_element_type=jnp.float32)
    w = jax.lax.dot_general(q, wk_ref[0], (((1,), (1,)), ((), ())),
                            preferred_element_type=jnp.float32)
    w_ref[0] = w


def _main_kernel(x_ref, w_ref, simt_ref, z_ref):
    s = pl.program_id(1)
    x = x_ref[0]
    w = w_ref[:, 0, :]
    simt = jax.lax.dot_general(w, x, (((1,), (1,)), ((), ())),
                               preferred_element_type=jnp.float32)
    simt_ref[0] = simt
    zc = jnp.dot(simt, x, preferred_element_type=jnp.float32)

    @pl.when(s == 0)
    def _():
        z_ref[0] = zc

    @pl.when(s > 0)
    def _():
        z_ref[0] += zc


def _att_kernel(z_ref, wv_ref, att_ref):
    att_ref[...] = jax.lax.dot_general(
        z_ref[...], wv_ref[...], (((2,), (1,)), ((1,), (0,))),
        preferred_element_type=jnp.float32)


def _topk_kernel(ns_ref, tv_ref, ti_ref, mask_ref):
    v = ns_ref[...]
    n_k = v.shape[1]
    kio = jax.lax.broadcasted_iota(jnp.int32, v.shape, 1)
    m0 = jnp.min(v, axis=1, keepdims=True)
    i0 = jnp.min(jnp.where(v == m0, kio, n_k), axis=1, keepdims=True)
    v1 = jnp.where(kio == i0, jnp.inf, v)
    m1 = jnp.min(v1, axis=1, keepdims=True)
    i1 = jnp.min(jnp.where(v1 == m1, kio, n_k), axis=1, keepdims=True)
    tv_ref[...] = jnp.concatenate([m0, m1], axis=1)
    ti_ref[...] = jnp.concatenate([i0, i1], axis=1)
    sel = (kio == i0) | (kio == i1)
    mask_ref[...] = jnp.broadcast_to(
        sel, mask_ref.shape).astype(jnp.float32)


def kernel(input, rim_hidden, Wq, Wk, Wv):
    B, S, D = input.shape
    K, H = rim_hidden.shape
    A = Wq.shape[2]
    BS = 512
    ns = S // BS

    h3 = rim_hidden.reshape(K, 1, H)

    w = pl.pallas_call(
        _prep_kernel,
        grid=(K,),
        in_specs=[
            pl.BlockSpec((1, 1, H), lambda k: (k, 0, 0)),
            pl.BlockSpec((1, H, A), lambda k: (k, 0, 0)),
            pl.BlockSpec((1, D, A), lambda k: (k, 0, 0)),
        ],
        out_specs=pl.BlockSpec((1, 1, D), lambda k: (k, 0, 0)),
        out_shape=jax.ShapeDtypeStruct((K, 1, D), jnp.float32),
    )(h3, Wq, Wk)

    simt, z = pl.pallas_call(
        _main_kernel,
        grid=(B, ns),
        in_specs=[
            pl.BlockSpec((1, BS, D), lambda b, s: (b, s, 0)),
            pl.BlockSpec((K, 1, D), lambda b, s: (0, 0, 0)),
        ],
        out_specs=[
            pl.BlockSpec((1, K, BS), lambda b, s: (b, 0, s)),
            pl.BlockSpec((1, K, D), lambda b, s: (b, 0, 0)),
        ],
        out_shape=[
            jax.ShapeDtypeStruct((B, K, S), jnp.float32),
            jax.ShapeDtypeStruct((B, K, D), jnp.float32),
        ],
    )(input, w)

    att_t = pl.pallas_call(
        _att_kernel,
        out_shape=jax.ShapeDtypeStruct((K, B, A), jnp.float32),
    )(z, Wv)
    attended = jnp.swapaxes(att_t, 0, 1)

    sim = jnp.concatenate(
        [simt, jnp.zeros((B, K, 1), jnp.float32)], axis=2)
    null_sim = jax.lax.slice(sim, (0, 0, S), (B, K, S + 1))

    topk_vals3, topk_idx3, update_mask = pl.pallas_call(
        _topk_kernel,
        out_shape=[
            jax.ShapeDtypeStruct((B, _ACT, 1), jnp.float32),
            jax.ShapeDtypeStruct((B, _ACT, 1), jnp.int32),
            jax.ShapeDtypeStruct((B, K, H), jnp.float32),
        ],
    )(null_sim)
    topk_vals = topk_vals3.reshape(B, _ACT)
    topk_idx = topk_idx3.reshape(B, _ACT)

    return (attended, sim, topk_vals, topk_idx, update_mask)


# speedup vs baseline: 2.4575x; 1.0156x over previous
import jax
import jax.numpy as jnp
from jax.experimental import pallas as pl
from jax.experimental.pallas import tpu as pltpu

_ACT = 2


def _prep_kernel(h_ref, wq_ref, wk_ref, w_ref):
    h = h_ref[0]
    q = jnp.dot(h, wq_ref[0], preferred_element_type=jnp.float32)
    w = jax.lax.dot_general(q, wk_ref[0], (((1,), (1,)), ((), ())),
                            preferred_element_type=jnp.float32)
    w_ref[0] = w


def _main_kernel(x_ref, w_ref, wv_ref, simt_ref, att_ref, z_ref):
    s = pl.program_id(1)
    ns = pl.num_programs(1)
    x = x_ref[0]
    w = w_ref[:, 0, :]
    simt = jax.lax.dot_general(w, x, (((1,), (1,)), ((), ())),
                               preferred_element_type=jnp.float32)
    simt_ref[0] = simt
    zc = jnp.dot(simt, x, preferred_element_type=jnp.float32)

    @pl.when(s == 0)
    def _():
        z_ref[...] = zc

    @pl.when(s > 0)
    def _():
        z_ref[...] += zc

    @pl.when(s == ns - 1)
    def _():
        z = z_ref[...]
        rows = [
            jnp.dot(z[k:k + 1, :], wv_ref[k],
                    preferred_element_type=jnp.float32)
            for k in range(z.shape[0])
        ]
        att_ref[0] = jnp.concatenate(rows, axis=0)


def _topk_kernel(ns_ref, tv_ref, ti_ref, mask_ref):
    v = ns_ref[...]
    n_k = v.shape[1]
    kio = jax.lax.broadcasted_iota(jnp.int32, v.shape, 1)
    m0 = jnp.min(v, axis=1, keepdims=True)
    i0 = jnp.min(jnp.where(v == m0, kio, n_k), axis=1, keepdims=True)
    v1 = jnp.where(kio == i0, jnp.inf, v)
    m1 = jnp.min(v1, axis=1, keepdims=True)
    i1 = jnp.min(jnp.where(v1 == m1, kio, n_k), axis=1, keepdims=True)
    tv_ref[...] = jnp.concatenate([m0, m1], axis=1)
    ti_ref[...] = jnp.concatenate([i0, i1], axis=1)
    sel = (kio == i0) | (kio == i1)
    mask_ref[...] = jnp.broadcast_to(
        sel, mask_ref.shape).astype(jnp.float32)


def kernel(input, rim_hidden, Wq, Wk, Wv):
    B, S, D = input.shape
    K, H = rim_hidden.shape
    A = Wq.shape[2]
    BS = 512
    ns = S // BS

    h3 = rim_hidden.reshape(K, 1, H)

    w = pl.pallas_call(
        _prep_kernel,
        grid=(K,),
        in_specs=[
            pl.BlockSpec((1, 1, H), lambda k: (k, 0, 0)),
            pl.BlockSpec((1, H, A), lambda k: (k, 0, 0)),
            pl.BlockSpec((1, D, A), lambda k: (k, 0, 0)),
        ],
        out_specs=pl.BlockSpec((1, 1, D), lambda k: (k, 0, 0)),
        out_shape=jax.ShapeDtypeStruct((K, 1, D), jnp.float32),
    )(h3, Wq, Wk)

    simt, att = pl.pallas_call(
        _main_kernel,
        grid=(B, ns),
        in_specs=[
            pl.BlockSpec((1, BS, D), lambda b, s: (b, s, 0)),
            pl.BlockSpec((K, 1, D), lambda b, s: (0, 0, 0)),
            pl.BlockSpec((K, D, A), lambda b, s: (0, 0, 0)),
        ],
        out_specs=[
            pl.BlockSpec((1, K, BS), lambda b, s: (b, 0, s)),
            pl.BlockSpec((1, K, A), lambda b, s: (b, 0, 0)),
        ],
        out_shape=[
            jax.ShapeDtypeStruct((B, K, S), jnp.float32),
            jax.ShapeDtypeStruct((B, K, A), jnp.float32),
        ],
        scratch_shapes=[pltpu.VMEM((K, D), jnp.float32)],
    )(input, w, Wv)

    sim = jnp.concatenate(
        [simt, jnp.zeros((B, K, 1), jnp.float32)], axis=2)
    null_sim = jax.lax.slice(sim, (0, 0, S), (B, K, S + 1))

    topk_vals3, topk_idx3, update_mask = pl.pallas_call(
        _topk_kernel,
        out_shape=[
            jax.ShapeDtypeStruct((B, _ACT, 1), jnp.float32),
            jax.ShapeDtypeStruct((B, _ACT, 1), jnp.int32),
            jax.ShapeDtypeStruct((B, K, H), jnp.float32),
        ],
    )(null_sim)
    topk_vals = topk_vals3.reshape(B, _ACT)
    topk_idx = topk_idx3.reshape(B, _ACT)

    return (att, sim, topk_vals, topk_idx, update_mask)
